# Initial kernel scaffold; baseline (speedup 1.0000x reference)
import functools

import jax
import jax.numpy as jnp
from jax import lax
from jax.experimental import pallas as pl
from jax.experimental.pallas import tpu as pltpu
from jax.experimental.pallas import tpu_sc as plsc

_N = 10000
_E = 320000
_D = 128

_NC = 2
_NS = 16
_NW = _NC * _NS
_EPW = _E // _NW
_CH = 80
_NCH = _EPW // _CH
_RPT = _N // _NS

_sc_mesh = plsc.VectorSubcoreMesh(core_axis_name="c", subcore_axis_name="s")


@functools.partial(
    pl.kernel,
    out_type=jax.ShapeDtypeStruct((_NC * _N, _D), jnp.float32),
    mesh=_sc_mesh,
    scratch_types=[
        pltpu.VMEM((_NCH, _CH), jnp.int32),
        pltpu.VMEM((_NCH, _CH), jnp.int32),
        pltpu.VMEM((_CH, _D), jnp.float32),
        pltpu.VMEM_SHARED((_N, _D), jnp.float32),
        pltpu.SemaphoreType.DMA,
    ],
)
def _sc_agg(x_hbm, src_hbm, dst_hbm, zeros_hbm, out_hbm,
            src_v, dst_v, rows_v, agg_sh, sem):
    cid = lax.axis_index("c")
    sid = lax.axis_index("s")
    wid = cid * _NS + sid
    row0 = sid * _RPT
    pltpu.sync_copy(zeros_hbm.at[pl.ds(row0, _RPT)],
                    agg_sh.at[pl.ds(row0, _RPT)])
    pltpu.sync_copy(src_hbm.at[wid], src_v)
    pltpu.sync_copy(dst_hbm.at[wid], dst_v)
    plsc.subcore_barrier()

    def body(j, carry):
        pltpu.async_copy(x_hbm.at[src_v.at[j]], rows_v, sem).wait()
        pltpu.sync_copy(rows_v, agg_sh.at[dst_v.at[j]], add=True)
        return carry

    lax.fori_loop(0, _NCH, body, 0)
    plsc.subcore_barrier()
    pltpu.sync_copy(agg_sh.at[pl.ds(row0, _RPT)],
                    out_hbm.at[pl.ds(cid * _N + row0, _RPT)])


_BLK = 2000
_GRID = _N // _BLK


def _mm_stats_body(x_ref, a0_ref, a1_ref, w_ref, b_ref, h_ref, st_ref):
    i = pl.program_id(0)
    s = x_ref[...] + a0_ref[...] + a1_ref[...]
    h = jnp.dot(s, w_ref[...], preferred_element_type=jnp.float32) + b_ref[...]
    h_ref[...] = h

    @pl.when(i == 0)
    def _():
        st_ref[...] = jnp.zeros_like(st_ref)

    su = jnp.sum(h, axis=0, keepdims=True)
    sq = jnp.sum(h * h, axis=0, keepdims=True)
    st_ref[...] += jnp.concatenate([su, sq], axis=0)


_mm_stats = pl.pallas_call(
    _mm_stats_body,
    grid=(_GRID,),
    in_specs=[
        pl.BlockSpec((_BLK, _D), lambda i: (i, 0)),
        pl.BlockSpec((_BLK, _D), lambda i: (i, 0)),
        pl.BlockSpec((_BLK, _D), lambda i: (i, 0)),
        pl.BlockSpec((_D, _D), lambda i: (0, 0)),
        pl.BlockSpec((1, _D), lambda i: (0, 0)),
    ],
    out_specs=[
        pl.BlockSpec((_BLK, _D), lambda i: (i, 0)),
        pl.BlockSpec((2, _D), lambda i: (0, 0)),
    ],
    out_shape=[
        jax.ShapeDtypeStruct((_N, _D), jnp.float32),
        jax.ShapeDtypeStruct((2, _D), jnp.float32),
    ],
)


def _norm_mm_body(h_ref, st_ref, g_ref, bt_ref, w_ref, b_ref, o_ref):
    mean = st_ref[0:1, :] * (1.0 / _N)
    var = st_ref[1:2, :] * (1.0 / _N) - mean * mean
    scale = g_ref[...] * lax.rsqrt(var + 128.0)
    hn = jnp.maximum((h_ref[...] - mean) * scale + bt_ref[...], 0.0)
    o_ref[...] = jnp.maximum(
        jnp.dot(hn, w_ref[...], preferred_element_type=jnp.float32)
        + b_ref[...], 0.0)


_norm_mm = pl.pallas_call(
    _norm_mm_body,
    grid=(_GRID,),
    in_specs=[
        pl.BlockSpec((_BLK, _D), lambda i: (i, 0)),
        pl.BlockSpec((2, _D), lambda i: (0, 0)),
        pl.BlockSpec((1, _D), lambda i: (0, 0)),
        pl.BlockSpec((1, _D), lambda i: (0, 0)),
        pl.BlockSpec((_D, _D), lambda i: (0, 0)),
        pl.BlockSpec((1, _D), lambda i: (0, 0)),
    ],
    out_specs=pl.BlockSpec((_BLK, _D), lambda i: (i, 0)),
    out_shape=jax.ShapeDtypeStruct((_N, _D), jnp.float32),
)


def _head_body(h_ref, w1_ref, b1_ref, w2_ref, b2_ref, o1_ref, o2_ref):
    t = jnp.maximum(
        jnp.dot(h_ref[...], w1_ref[...], preferred_element_type=jnp.float32)
        + b1_ref[...], 0.0)
    z = (jnp.dot(t, w2_ref[...], preferred_element_type=jnp.float32)
         + b2_ref[...])
    o1_ref[...] = z
    m = jnp.max(z, axis=1, keepdims=True)
    e = jnp.exp(z - m)
    o2_ref[...] = e / jnp.sum(e, axis=1, keepdims=True)


_head = pl.pallas_call(
    _head_body,
    grid=(_GRID,),
    in_specs=[
        pl.BlockSpec((_BLK, 3 * _D), lambda i: (i, 0)),
        pl.BlockSpec((3 * _D, 3 * _D), lambda i: (0, 0)),
        pl.BlockSpec((1, 3 * _D), lambda i: (0, 0)),
        pl.BlockSpec((3 * _D, _D), lambda i: (0, 0)),
        pl.BlockSpec((1, _D), lambda i: (0, 0)),
    ],
    out_specs=[
        pl.BlockSpec((_BLK, _D), lambda i: (i, 0)),
        pl.BlockSpec((_BLK, _D), lambda i: (i, 0)),
    ],
    out_shape=[
        jax.ShapeDtypeStruct((_N, _D), jnp.float32),
        jax.ShapeDtypeStruct((_N, _D), jnp.float32),
    ],
)


def kernel(x, edge_index,
           c1_W1, c1_b1, c1_g, c1_bt, c1_W2, c1_b2,
           c2_W1, c2_b1, c2_g, c2_bt, c2_W2, c2_b2,
           c3_W1, c3_b1, c3_g, c3_bt, c3_W2, c3_b2,
           lin1_W, lin1_b, lin2_W, lin2_b):
    src3 = edge_index[0].reshape(_NW, _NCH, _CH)
    dst3 = edge_index[1].reshape(_NW, _NCH, _CH)
    zeros = jnp.zeros((_N, _D), jnp.float32)

    def conv(xin, W1, b1, g, bt, W2, b2):
        agg = _sc_agg(xin, src3, dst3, zeros)
        hpre, st = _mm_stats(xin, agg[:_N], agg[_N:], W1, b1.reshape(1, _D))
        return _norm_mm(hpre, st, g.reshape(1, _D), bt.reshape(1, _D), W2,
                        b2.reshape(1, _D))

    h1 = conv(x, c1_W1, c1_b1, c1_g, c1_bt, c1_W2, c1_b2)
    h2 = conv(h1, c2_W1, c2_b1, c2_g, c2_bt, c2_W2, c2_b2)
    h3 = conv(h2, c3_W1, c3_b1, c3_g, c3_bt, c3_W2, c3_b2)
    hcat = jnp.concatenate([h1, h2, h3], axis=1)
    return tuple(_head(hcat, lin1_W, lin1_b.reshape(1, 3 * _D),
                       lin2_W, lin2_b.reshape(1, _D)))


# speedup vs baseline: 5.6995x; 5.6995x over previous
import functools

import jax
import jax.numpy as jnp
from jax import lax
from jax.experimental import pallas as pl
from jax.experimental.pallas import tpu as pltpu
from jax.experimental.pallas import tpu_sc as plsc

_N = 10000
_E = 320000
_D = 128

_NC = 2
_NS = 16
_NW = _NC * _NS
_EPW = _E // _NW
_CH = 80
_NCH = _EPW // _CH
_NP = 10240
_RPT = _NP // _NS

@functools.cache
def _make_sc_agg():
    mesh = plsc.VectorSubcoreMesh(core_axis_name="c", subcore_axis_name="s")

    @functools.partial(
        pl.kernel,
        out_type=jax.ShapeDtypeStruct((_NC * _NP, _D), jnp.float32),
        mesh=mesh,
        scratch_types=[
            pltpu.VMEM((_NCH, _CH), jnp.int32),
            pltpu.VMEM((_NCH, _CH), jnp.int32),
            pltpu.VMEM((_CH, _D), jnp.float32),
            pltpu.VMEM_SHARED((_NP, _D), jnp.float32),
            pltpu.SemaphoreType.DMA,
        ],
    )
    def sc_agg(x_hbm, src_hbm, dst_hbm, zeros_hbm, out_hbm,
               src_v, dst_v, rows_v, agg_sh, sem):
        cid = lax.axis_index("c")
        sid = lax.axis_index("s")
        wid = cid * _NS + sid
        row0 = sid * _RPT
        pltpu.sync_copy(zeros_hbm.at[pl.ds(row0, _RPT)],
                        agg_sh.at[pl.ds(row0, _RPT)])
        pltpu.sync_copy(src_hbm.at[wid], src_v)
        pltpu.sync_copy(dst_hbm.at[wid], dst_v)
        plsc.subcore_barrier()

        def body(j, carry):
            pltpu.async_copy(x_hbm.at[src_v.at[j]], rows_v, sem).wait()
            pltpu.sync_copy(rows_v, agg_sh.at[dst_v.at[j]], add=True)
            return carry

        lax.fori_loop(0, _NCH, body, 0)
        plsc.subcore_barrier()
        pltpu.sync_copy(agg_sh.at[pl.ds(row0, _RPT)],
                        out_hbm.at[pl.ds(cid * _NP + row0, _RPT)])

    return sc_agg


def _sc_agg(x, src3, dst3, zeros):
    return _make_sc_agg()(x, src3, dst3, zeros)


_BLK = 2000
_GRID = _N // _BLK


def _mm_stats_body(x_ref, a0_ref, a1_ref, w_ref, b_ref, h_ref, st_ref):
    i = pl.program_id(0)
    s = x_ref[...] + a0_ref[...] + a1_ref[...]
    h = jnp.dot(s, w_ref[...], preferred_element_type=jnp.float32) + b_ref[...]
    h_ref[...] = h

    @pl.when(i == 0)
    def _():
        st_ref[...] = jnp.zeros_like(st_ref)

    su = jnp.sum(h, axis=0, keepdims=True)
    sq = jnp.sum(h * h, axis=0, keepdims=True)
    st_ref[...] += jnp.concatenate([su, sq], axis=0)


_mm_stats = pl.pallas_call(
    _mm_stats_body,
    grid=(_GRID,),
    in_specs=[
        pl.BlockSpec((_BLK, _D), lambda i: (i, 0)),
        pl.BlockSpec((_BLK, _D), lambda i: (i, 0)),
        pl.BlockSpec((_BLK, _D), lambda i: (i, 0)),
        pl.BlockSpec((_D, _D), lambda i: (0, 0)),
        pl.BlockSpec((1, _D), lambda i: (0, 0)),
    ],
    out_specs=[
        pl.BlockSpec((_BLK, _D), lambda i: (i, 0)),
        pl.BlockSpec((2, _D), lambda i: (0, 0)),
    ],
    out_shape=[
        jax.ShapeDtypeStruct((_N, _D), jnp.float32),
        jax.ShapeDtypeStruct((2, _D), jnp.float32),
    ],
)


def _norm_mm_body(h_ref, st_ref, g_ref, bt_ref, w_ref, b_ref, o_ref):
    mean = st_ref[0:1, :] * (1.0 / _N)
    var = st_ref[1:2, :] * (1.0 / _N) - mean * mean
    scale = g_ref[...] * lax.rsqrt(var + 128.0)
    hn = jnp.maximum((h_ref[...] - mean) * scale + bt_ref[...], 0.0)
    o_ref[...] = jnp.maximum(
        jnp.dot(hn, w_ref[...], preferred_element_type=jnp.float32)
        + b_ref[...], 0.0)


_norm_mm = pl.pallas_call(
    _norm_mm_body,
    grid=(_GRID,),
    in_specs=[
        pl.BlockSpec((_BLK, _D), lambda i: (i, 0)),
        pl.BlockSpec((2, _D), lambda i: (0, 0)),
        pl.BlockSpec((1, _D), lambda i: (0, 0)),
        pl.BlockSpec((1, _D), lambda i: (0, 0)),
        pl.BlockSpec((_D, _D), lambda i: (0, 0)),
        pl.BlockSpec((1, _D), lambda i: (0, 0)),
    ],
    out_specs=pl.BlockSpec((_BLK, _D), lambda i: (i, 0)),
    out_shape=jax.ShapeDtypeStruct((_N, _D), jnp.float32),
)


def _head_body(h_ref, w1_ref, b1_ref, w2_ref, b2_ref, o1_ref, o2_ref):
    t = jnp.maximum(
        jnp.dot(h_ref[...], w1_ref[...], preferred_element_type=jnp.float32)
        + b1_ref[...], 0.0)
    z = (jnp.dot(t, w2_ref[...], preferred_element_type=jnp.float32)
         + b2_ref[...])
    o1_ref[...] = z
    m = jnp.max(z, axis=1, keepdims=True)
    e = jnp.exp(z - m)
    o2_ref[...] = e / jnp.sum(e, axis=1, keepdims=True)


_head = pl.pallas_call(
    _head_body,
    grid=(_GRID,),
    in_specs=[
        pl.BlockSpec((_BLK, 3 * _D), lambda i: (i, 0)),
        pl.BlockSpec((3 * _D, 3 * _D), lambda i: (0, 0)),
        pl.BlockSpec((1, 3 * _D), lambda i: (0, 0)),
        pl.BlockSpec((3 * _D, _D), lambda i: (0, 0)),
        pl.BlockSpec((1, _D), lambda i: (0, 0)),
    ],
    out_specs=[
        pl.BlockSpec((_BLK, _D), lambda i: (i, 0)),
        pl.BlockSpec((_BLK, _D), lambda i: (i, 0)),
    ],
    out_shape=[
        jax.ShapeDtypeStruct((_N, _D), jnp.float32),
        jax.ShapeDtypeStruct((_N, _D), jnp.float32),
    ],
)


def kernel(x, edge_index,
           c1_W1, c1_b1, c1_g, c1_bt, c1_W2, c1_b2,
           c2_W1, c2_b1, c2_g, c2_bt, c2_W2, c2_b2,
           c3_W1, c3_b1, c3_g, c3_bt, c3_W2, c3_b2,
           lin1_W, lin1_b, lin2_W, lin2_b):
    src3 = edge_index[0].reshape(_NW, _NCH, _CH)
    dst3 = edge_index[1].reshape(_NW, _NCH, _CH)
    zeros = jnp.zeros((_NP, _D), jnp.float32)

    def conv(xin, W1, b1, g, bt, W2, b2):
        agg = _sc_agg(xin, src3, dst3, zeros)
        hpre, st = _mm_stats(xin, agg[:_N], agg[_NP:_NP + _N], W1,
                             b1.reshape(1, _D))
        return _norm_mm(hpre, st, g.reshape(1, _D), bt.reshape(1, _D), W2,
                        b2.reshape(1, _D))

    h1 = conv(x, c1_W1, c1_b1, c1_g, c1_bt, c1_W2, c1_b2)
    h2 = conv(h1, c2_W1, c2_b1, c2_g, c2_bt, c2_W2, c2_b2)
    h3 = conv(h2, c3_W1, c3_b1, c3_g, c3_bt, c3_W2, c3_b2)
    hcat = jnp.concatenate([h1, h2, h3], axis=1)
    return tuple(_head(hcat, lin1_W, lin1_b.reshape(1, 3 * _D),
                       lin2_W, lin2_b.reshape(1, _D)))
